# Initial kernel scaffold; baseline (speedup 1.0000x reference)
import jax
import jax.numpy as jnp
from jax.experimental import pallas as pl


def kernel(x, w0, w, Z):
    raise NotImplementedError("write your pallas kernel here")


# speedup vs baseline: 1.7835x; 1.7835x over previous
import functools

import jax
import jax.numpy as jnp
from jax import lax
from jax.experimental import pallas as pl
from jax.experimental.pallas import tpu as pltpu
from jax.experimental.pallas import tpu_sc as plsc

_N = 100000
_D = 64
_F = 26
_B = 4096
_ALPHA = 0.001
_BETA = 0.001

_NC = 2
_NS = 16
_NW = _NC * _NS
_SPW = _B // _NW
_CS = 16
_NCH = _SPW // _CS
_IPC = _CS * _F
_ICOLS = 104
_IROWS = _IPC // _ICOLS
_LANES = 16


def _fis_body(xidx_hbm, xwf_hbm, w0_hbm, w_hbm, z_hbm, y_hbm, part_hbm,
              idx_a, idx_b, rows_a, rows_b, widx, wvals, ybuf, pbuf, w0buf,
              sem_ia, sem_ib, sem_ga, sem_gb, sem_w):
    wid = lax.axis_index("s") * _NC + lax.axis_index("c")
    irow0 = wid * (_NCH * _IROWS)

    idx_bufs = (idx_a, idx_b)
    row_bufs = (rows_a, rows_b)
    isems = (sem_ia, sem_ib)
    gsems = (sem_ga, sem_gb)

    def start_idx(c):
        return pltpu.async_copy(
            xidx_hbm.at[pl.ds(irow0 + c * _IROWS, _IROWS)],
            idx_bufs[c % 2], isems[c % 2])

    def start_gathers(c):
        return [pltpu.async_copy(
                    z_hbm.at[idx_bufs[c % 2].at[j]],
                    row_bufs[c % 2].at[pl.ds(j * _ICOLS, _ICOLS)],
                    gsems[c % 2])
                for j in range(_IROWS)]

    start_idx(0).wait()
    gh = start_gathers(0)
    pltpu.sync_copy(xwf_hbm.at[wid], widx)
    wh = [pltpu.async_copy(w_hbm.at[widx.at[f]], wvals.at[f], sem_w)
          for f in range(_F)]
    ih = start_idx(1)
    pltpu.sync_copy(w0_hbm, w0buf)
    for h in wh:
        h.wait()
    w0s = w0buf[...]
    lanes = lax.iota(jnp.int32, _LANES)

    qsum = jnp.float32(0.0)
    sqw = jnp.zeros((_LANES,), jnp.float32)

    for c in range(_NCH):
        rows = row_bufs[c % 2]
        for h in gh:
            h.wait()
        if c + 1 < _NCH:
            ih.wait()
            gh = start_gathers(c + 1)
        if c + 2 < _NCH:
            ih = start_idx(c + 2)

        def sample_body(i, carry, rows=rows):
            qacc, yvec = carry
            r0 = i * _F
            s0 = jnp.zeros((_LANES,), jnp.float32)
            s1 = jnp.zeros((_LANES,), jnp.float32)
            s2 = jnp.zeros((_LANES,), jnp.float32)
            s3 = jnp.zeros((_LANES,), jnp.float32)
            q = jnp.zeros((_LANES,), jnp.float32)
            for f in range(_F):
                r = r0 + f
                z0 = rows[r, pl.ds(0 * _LANES, _LANES)]
                z1 = rows[r, pl.ds(1 * _LANES, _LANES)]
                z2 = rows[r, pl.ds(2 * _LANES, _LANES)]
                z3 = rows[r, pl.ds(3 * _LANES, _LANES)]
                s0 = s0 + z0
                s1 = s1 + z1
                s2 = s2 + z2
                s3 = s3 + z3
                q = q + z0 * z0 + z1 * z1 + z2 * z2 + z3 * z3
            sv = s0 * s0 + s1 * s1 + s2 * s2 + s3 * s3
            q_s = jnp.sum(q)
            s_s = jnp.sum(sv)
            yvec = jnp.where(lanes == i, 0.5 * (s_s - q_s), yvec)
            return qacc + q_s, yvec

        qsum, yv = lax.fori_loop(
            0, _CS, sample_body,
            (qsum, jnp.zeros((_LANES,), jnp.float32)))

        lw = jnp.zeros((_LANES,), jnp.float32)
        for f in range(_F):
            v = wvals[f, pl.ds(c * _CS, _CS)]
            lw = lw + v
            sqw = sqw + v * v
        ybuf[pl.ds(c * _CS, _CS)] = yv + lw + w0s

    p = _ALPHA * jnp.sum(sqw) + _BETA * qsum
    pbuf[...] = jnp.zeros((_LANES,), jnp.float32) + p
    pltpu.sync_copy(ybuf, y_hbm.at[pl.ds(wid * _SPW, _SPW)])
    pltpu.sync_copy(pbuf, part_hbm.at[wid])


_fis_call = functools.partial(
    pl.kernel,
    out_type=(jax.ShapeDtypeStruct((_B,), jnp.float32),
              jax.ShapeDtypeStruct((_NW, _LANES), jnp.float32)),
    mesh=plsc.VectorSubcoreMesh(core_axis_name="c", subcore_axis_name="s"),
    compiler_params=pltpu.CompilerParams(
        needs_layout_passes=False, use_tc_tiling_on_sc=False),
    scratch_types=[
        pltpu.VMEM((_IROWS, _ICOLS), jnp.int32),
        pltpu.VMEM((_IROWS, _ICOLS), jnp.int32),
        pltpu.VMEM((_IPC, _D), jnp.float32),
        pltpu.VMEM((_IPC, _D), jnp.float32),
        pltpu.VMEM((_F, _SPW), jnp.int32),
        pltpu.VMEM((_F, _SPW), jnp.float32),
        pltpu.VMEM((_SPW,), jnp.float32),
        pltpu.VMEM((_LANES,), jnp.float32),
        pltpu.VMEM((_LANES,), jnp.float32),
        pltpu.SemaphoreType.DMA,
        pltpu.SemaphoreType.DMA,
        pltpu.SemaphoreType.DMA,
        pltpu.SemaphoreType.DMA,
        pltpu.SemaphoreType.DMA,
    ],
)(_fis_body)


def kernel(x, w0, w, Z):
    x = x.astype(jnp.int32)
    xidx = x.reshape(_B * _F // _ICOLS, _ICOLS)
    xwf = jnp.swapaxes(x.T.reshape(_F, _NW, _SPW), 0, 1)
    w0v = jnp.broadcast_to(w0, (_LANES,))
    y, part = _fis_call(xidx, xwf, w0v, w, Z)
    return y, jnp.sum(part[:, 0])


# speedup vs baseline: 1.7880x; 1.0025x over previous
import functools

import jax
import jax.numpy as jnp
from jax import lax
from jax.experimental import pallas as pl
from jax.experimental.pallas import tpu as pltpu
from jax.experimental.pallas import tpu_sc as plsc

_N = 100000
_D = 64
_F = 26
_B = 4096
_ALPHA = 0.001
_BETA = 0.001

_NC = 2
_NS = 16
_NW = _NC * _NS
_SPW = _B // _NW
_CS = 16
_NCH = _SPW // _CS
_IPC = _CS * _F
_LANES = 16


def _fis_body(x_hbm, w0_hbm, w_hbm, z_hbm, y_hbm, part_hbm,
              idx_a, idx_b, rows_a, rows_b, wch_a, wch_b, ybuf, pbuf, w0buf,
              sem_ia, sem_ib, sem_ga, sem_gb, sem_wa, sem_wb):
    wid = lax.axis_index("s") * _NC + lax.axis_index("c")
    samp0 = wid * _SPW

    idx_bufs = (idx_a, idx_b)
    row_bufs = (rows_a, rows_b)
    wch_bufs = (wch_a, wch_b)
    isems = (sem_ia, sem_ib)
    gsems = (sem_ga, sem_gb)
    wsems = (sem_wa, sem_wb)

    def start_idx(c):
        return pltpu.async_copy(
            x_hbm.at[pl.ds(samp0 + c * _CS, _CS)],
            idx_bufs[c % 2], isems[c % 2])

    def start_gathers(c):
        idx = idx_bufs[c % 2]
        rows = row_bufs[c % 2]
        wch = wch_bufs[c % 2]
        hs = []
        for i in range(_CS):
            hs.append(pltpu.async_copy(
                z_hbm.at[idx.at[i]], rows.at[pl.ds(i * _F, _F)],
                gsems[c % 2]))
        for i in range(_CS):
            hs.append(pltpu.async_copy(
                w_hbm.at[idx.at[i]], wch.at[i], wsems[c % 2]))
        return hs

    start_idx(0).wait()
    gh = start_gathers(0)
    ih = start_idx(1)
    pltpu.sync_copy(w0_hbm, w0buf)
    w0s = w0buf[...]
    lanes = lax.iota(jnp.int32, _LANES)

    qsum = jnp.float32(0.0)
    sqw = jnp.zeros((_LANES,), jnp.float32)

    for c in range(_NCH):
        rows = row_bufs[c % 2]
        wch = wch_bufs[c % 2]
        for h in gh:
            h.wait()
        if c + 1 < _NCH:
            ih.wait()
            gh = start_gathers(c + 1)
        if c + 2 < _NCH:
            ih = start_idx(c + 2)

        def sample_body(i, carry, rows=rows):
            qacc, yvec = carry
            r0 = i * _F
            s0 = jnp.zeros((_LANES,), jnp.float32)
            s1 = jnp.zeros((_LANES,), jnp.float32)
            s2 = jnp.zeros((_LANES,), jnp.float32)
            s3 = jnp.zeros((_LANES,), jnp.float32)
            q = jnp.zeros((_LANES,), jnp.float32)
            for f in range(_F):
                r = r0 + f
                z0 = rows[r, pl.ds(0 * _LANES, _LANES)]
                z1 = rows[r, pl.ds(1 * _LANES, _LANES)]
                z2 = rows[r, pl.ds(2 * _LANES, _LANES)]
                z3 = rows[r, pl.ds(3 * _LANES, _LANES)]
                s0 = s0 + z0
                s1 = s1 + z1
                s2 = s2 + z2
                s3 = s3 + z3
                q = q + z0 * z0 + z1 * z1 + z2 * z2 + z3 * z3
            sv = s0 * s0 + s1 * s1 + s2 * s2 + s3 * s3
            q_s = jnp.sum(q)
            s_s = jnp.sum(sv)
            yvec = jnp.where(lanes == i, 0.5 * (s_s - q_s), yvec)
            return qacc + q_s, yvec

        qsum, yv = lax.fori_loop(
            0, _CS, sample_body,
            (qsum, jnp.zeros((_LANES,), jnp.float32)))

        lw = jnp.zeros((_LANES,), jnp.float32)
        for f in range(_F):
            v = plsc.load_gather(wch, [lanes, jnp.full((_LANES,), f, jnp.int32)])
            lw = lw + v
            sqw = sqw + v * v
        ybuf[pl.ds(c * _CS, _CS)] = yv + lw + w0s

    p = _ALPHA * jnp.sum(sqw) + _BETA * qsum
    pbuf[...] = jnp.zeros((_LANES,), jnp.float32) + p
    pltpu.sync_copy(ybuf, y_hbm.at[pl.ds(wid * _SPW, _SPW)])
    pltpu.sync_copy(pbuf, part_hbm.at[wid])


_fis_call = functools.partial(
    pl.kernel,
    out_type=(jax.ShapeDtypeStruct((_B,), jnp.float32),
              jax.ShapeDtypeStruct((_NW, _LANES), jnp.float32)),
    mesh=plsc.VectorSubcoreMesh(core_axis_name="c", subcore_axis_name="s"),
    compiler_params=pltpu.CompilerParams(
        needs_layout_passes=False, use_tc_tiling_on_sc=False),
    scratch_types=[
        pltpu.VMEM((_CS, _F), jnp.int32),
        pltpu.VMEM((_CS, _F), jnp.int32),
        pltpu.VMEM((_IPC, _D), jnp.float32),
        pltpu.VMEM((_IPC, _D), jnp.float32),
        pltpu.VMEM((_CS, _F), jnp.float32),
        pltpu.VMEM((_CS, _F), jnp.float32),
        pltpu.VMEM((_SPW,), jnp.float32),
        pltpu.VMEM((_LANES,), jnp.float32),
        pltpu.VMEM((_LANES,), jnp.float32),
        pltpu.SemaphoreType.DMA,
        pltpu.SemaphoreType.DMA,
        pltpu.SemaphoreType.DMA,
        pltpu.SemaphoreType.DMA,
        pltpu.SemaphoreType.DMA,
        pltpu.SemaphoreType.DMA,
    ],
)(_fis_body)


def kernel(x, w0, w, Z):
    x = x.astype(jnp.int32)
    w0v = jnp.broadcast_to(w0, (_LANES,))
    y, part = _fis_call(x, w0v, w, Z)
    return y, jnp.sum(part[:, 0])


# speedup vs baseline: 1.8650x; 1.0431x over previous
import functools

import jax
import jax.numpy as jnp
from jax import lax
from jax.experimental import pallas as pl
from jax.experimental.pallas import tpu as pltpu
from jax.experimental.pallas import tpu_sc as plsc

_N = 100000
_D = 64
_F = 26
_B = 4096
_ALPHA = 0.001
_BETA = 0.001

_NC = 2
_NS = 16
_NW = _NC * _NS
_SPW = _B // _NW
_CS = 16
_NCH = _SPW // _CS
_IPC = _CS * _F
_IPW = _SPW * _F
_ICOLS = 104
_IROWS = _IPC // _ICOLS
_LANES = 16


def _fis_body(x_hbm, w0_hbm, w_hbm, z_hbm, y_hbm, part_hbm,
              idxbuf, rows_a, rows_b, wch_a, wch_b, ybuf, pbuf, w0buf,
              sem_ga, sem_gb, sem_wa, sem_wb):
    wid = lax.axis_index("s") * _NC + lax.axis_index("c")

    row_bufs = (rows_a, rows_b)
    wch_bufs = (wch_a, wch_b)
    gsems = (sem_ga, sem_gb)
    wsems = (sem_wa, sem_wb)

    def start_gathers(c):
        rows = row_bufs[c % 2]
        wch = wch_bufs[c % 2]
        hs = []
        for j in range(_IROWS):
            idx = idxbuf.at[pl.ds(c * _IPC + j * _ICOLS, _ICOLS)]
            hs.append(pltpu.async_copy(
                z_hbm.at[idx], rows.at[pl.ds(j * _ICOLS, _ICOLS)],
                gsems[c % 2]))
            hs.append(pltpu.async_copy(
                w_hbm.at[idx], wch.at[pl.ds(j * _ICOLS, _ICOLS)],
                wsems[c % 2]))
        return hs

    pltpu.sync_copy(x_hbm.at[pl.ds(wid * _IPW, _IPW)], idxbuf)
    gh = start_gathers(0)
    nh = start_gathers(1)
    pltpu.sync_copy(w0_hbm, w0buf)
    w0s = w0buf[...]
    lanes = lax.iota(jnp.int32, _LANES)

    qsum = jnp.float32(0.0)
    sqw = jnp.zeros((_LANES,), jnp.float32)

    for c in range(_NCH):
        rows = row_bufs[c % 2]
        wch = wch_bufs[c % 2]
        for h in gh:
            h.wait()
        gh = nh
        if c + 2 < _NCH:
            nh = start_gathers(c + 2)

        def sample_body(i, carry, rows=rows):
            qacc, yvec = carry
            r0 = i * _F
            s0 = jnp.zeros((_LANES,), jnp.float32)
            s1 = jnp.zeros((_LANES,), jnp.float32)
            s2 = jnp.zeros((_LANES,), jnp.float32)
            s3 = jnp.zeros((_LANES,), jnp.float32)
            q = jnp.zeros((_LANES,), jnp.float32)
            for f in range(_F):
                r = r0 + f
                z0 = rows[r, pl.ds(0 * _LANES, _LANES)]
                z1 = rows[r, pl.ds(1 * _LANES, _LANES)]
                z2 = rows[r, pl.ds(2 * _LANES, _LANES)]
                z3 = rows[r, pl.ds(3 * _LANES, _LANES)]
                s0 = s0 + z0
                s1 = s1 + z1
                s2 = s2 + z2
                s3 = s3 + z3
                q = q + z0 * z0 + z1 * z1 + z2 * z2 + z3 * z3
            sv = s0 * s0 + s1 * s1 + s2 * s2 + s3 * s3
            q_s = jnp.sum(q)
            s_s = jnp.sum(sv)
            yvec = jnp.where(lanes == i, 0.5 * (s_s - q_s), yvec)
            return qacc + q_s, yvec

        qsum, yv = lax.fori_loop(
            0, _CS, sample_body,
            (qsum, jnp.zeros((_LANES,), jnp.float32)))

        lw = jnp.zeros((_LANES,), jnp.float32)
        fidx = lanes * _F
        for f in range(_F):
            v = plsc.load_gather(wch, [fidx + f])
            lw = lw + v
            sqw = sqw + v * v
        ybuf[pl.ds(c * _CS, _CS)] = yv + lw + w0s

    p = _ALPHA * jnp.sum(sqw) + _BETA * qsum
    pbuf[...] = jnp.zeros((_LANES,), jnp.float32) + p
    pltpu.sync_copy(ybuf, y_hbm.at[pl.ds(wid * _SPW, _SPW)])
    pltpu.sync_copy(pbuf, part_hbm.at[wid])


_fis_call = functools.partial(
    pl.kernel,
    out_type=(jax.ShapeDtypeStruct((_B,), jnp.float32),
              jax.ShapeDtypeStruct((_NW, _LANES), jnp.float32)),
    mesh=plsc.VectorSubcoreMesh(core_axis_name="c", subcore_axis_name="s"),
    compiler_params=pltpu.CompilerParams(
        needs_layout_passes=False, use_tc_tiling_on_sc=False),
    scratch_types=[
        pltpu.VMEM((_IPW,), jnp.int32),
        pltpu.VMEM((_IPC, _D), jnp.float32),
        pltpu.VMEM((_IPC, _D), jnp.float32),
        pltpu.VMEM((_IPC,), jnp.float32),
        pltpu.VMEM((_IPC,), jnp.float32),
        pltpu.VMEM((_SPW,), jnp.float32),
        pltpu.VMEM((_LANES,), jnp.float32),
        pltpu.VMEM((_LANES,), jnp.float32),
        pltpu.SemaphoreType.DMA,
        pltpu.SemaphoreType.DMA,
        pltpu.SemaphoreType.DMA,
        pltpu.SemaphoreType.DMA,
    ],
)(_fis_body)


def kernel(x, w0, w, Z):
    xf = x.astype(jnp.int32).reshape(_B * _F)
    w0v = jnp.broadcast_to(w0, (_LANES,))
    y, part = _fis_call(xf, w0v, w, Z)
    return y, jnp.sum(part[:, 0])
